# baseline (speedup 1.0000x reference)
import jax
import jax.numpy as jnp
from jax.experimental import pallas as pl

_B, _H, _W, _C = 32, 14, 14, 512
_KM_ITERS = 11


def _points_body(x_ref, pts_ref):
    x = x_ref[0]
    colmax = jnp.max(x, axis=0)
    arg_w = jnp.argmax(colmax, axis=0)
    rowmax = jnp.max(x, axis=1)
    arg_h = jnp.argmax(rowmax, axis=0)
    pts_ref[0, 0, :] = arg_w.astype(jnp.float32)
    pts_ref[0, 1, :] = arg_h.astype(jnp.float32)


def _kmeans_body(pts_ref, perm_ref, mask_ref):
    px = pts_ref[:, 0, :]
    py = pts_ref[:, 1, :]
    P = perm_ref[...]
    cx = jnp.dot(P, px[:, 0:2], preferred_element_type=jnp.float32)
    cy = jnp.dot(P, py[:, 0:2], preferred_element_type=jnp.float32)
    c0x, c1x = cx[:, 0:1], cx[:, 1:2]
    c0y, c1y = cy[:, 0:1], cy[:, 1:2]
    m1 = jnp.zeros((_B, _C), jnp.float32)
    for _ in range(_KM_ITERS):
        d0 = (px - c0x) ** 2 + (py - c0y) ** 2
        d1 = (px - c1x) ** 2 + (py - c1y) ** 2
        m1 = (d1 < d0).astype(jnp.float32)
        m0 = 1.0 - m1
        s1 = jnp.sum(m1, axis=1, keepdims=True)
        cnt1 = jnp.maximum(s1, 1.0)
        cnt0 = jnp.maximum(jnp.float32(_C) - s1, 1.0)
        c0x = jnp.sum(px * m1, axis=1, keepdims=True) / cnt1
        c0y = jnp.sum(py * m1, axis=1, keepdims=True) / cnt1
        c1x = jnp.sum(px * m0, axis=1, keepdims=True) / cnt0
        c1y = jnp.sum(py * m0, axis=1, keepdims=True) / cnt0
    mask_ref[...] = m1[:, None, :]


def _mask_body(x_ref, m_ref, c0_ref, c1_ref):
    x = x_ref[0]
    m = m_ref[0, 0, :] > 0.0
    c1_ref[0] = jnp.where(m, x, 0.0)
    c0_ref[0] = jnp.where(m, 0.0, x)


def kernel(feature_batch):
    pts = pl.pallas_call(
        _points_body,
        grid=(_B,),
        in_specs=[pl.BlockSpec((1, _H, _W, _C), lambda i: (i, 0, 0, 0))],
        out_specs=pl.BlockSpec((1, 2, _C), lambda i: (i, 0, 0)),
        out_shape=jax.ShapeDtypeStruct((_B, 2, _C), jnp.float32),
    )(feature_batch)

    perm = jax.random.permutation(jax.random.key(1), _B)
    P = jax.nn.one_hot(perm, _B, dtype=jnp.float32)

    mask = pl.pallas_call(
        _kmeans_body,
        out_shape=jax.ShapeDtypeStruct((_B, 1, _C), jnp.float32),
    )(pts, P)

    c0, c1 = pl.pallas_call(
        _mask_body,
        grid=(_B,),
        in_specs=[pl.BlockSpec((1, _H, _W, _C), lambda i: (i, 0, 0, 0)),
                  pl.BlockSpec((1, 1, _C), lambda i: (i, 0, 0))],
        out_specs=[pl.BlockSpec((1, _H, _W, _C), lambda i: (i, 0, 0, 0)),
                   pl.BlockSpec((1, _H, _W, _C), lambda i: (i, 0, 0, 0))],
        out_shape=[jax.ShapeDtypeStruct((_B, _H, _W, _C), jnp.float32),
                   jax.ShapeDtypeStruct((_B, _H, _W, _C), jnp.float32)],
    )(feature_batch, mask)
    return (c0, c1)
